# baseline (speedup 1.0000x reference)
import functools

import jax
import jax.numpy as jnp
from jax import lax
from jax.experimental import pallas as pl
from jax.experimental.pallas import tpu as pltpu
from jax.experimental.pallas import tpu_sc as plsc

N = 100000
E = 1600000
G = 256
H = 128
B = 1000
NB = N // B
GW = 24
GP = G + 16
EPS = 1e-5


def _onehot(bg, g0):
    return (bg[:, None] - g0 == lax.broadcasted_iota(jnp.int32, (B, GW), 1)).astype(jnp.float32)


def _wbase(bg):
    return (bg[0] // 8) * 8


def _segdot(m, t):
    return lax.dot_general(m, t, (((0,), (0,)), ((), ())),
                           precision=lax.Precision.HIGHEST,
                           preferred_element_type=jnp.float32)


def _leaky(x):
    return jnp.where(x >= 0, x, 0.01 * x)


def _p0_body(x_ref, w_ref, b3_ref, u_ref, cnt_ref):
    i = pl.program_id(0)

    @pl.when(i == 0)
    def _():
        cnt_ref[...] = jnp.zeros_like(cnt_ref)

    u_ref[...] = jnp.dot(x_ref[...], w_ref[...], preferred_element_type=jnp.float32)
    bg = b3_ref[0, 0, :]
    g0 = _wbase(bg)
    m = _onehot(bg, g0)
    cnt_ref[pl.ds(g0, GW), :] += _segdot(m, jnp.ones((B, H), jnp.float32))


def _p0(xpad, w1pad, batch3):
    return pl.pallas_call(
        _p0_body,
        grid=(NB,),
        in_specs=[
            pl.BlockSpec((B, H), lambda i: (i, 0)),
            pl.BlockSpec((H, H), lambda i: (0, 0)),
            pl.BlockSpec((1, 1, B), lambda i: (i, 0, 0)),
        ],
        out_specs=[
            pl.BlockSpec((B, H), lambda i: (i, 0)),
            pl.BlockSpec((GP, H), lambda i: (0, 0)),
        ],
        out_shape=[
            jax.ShapeDtypeStruct((N, H), jnp.float32),
            jax.ShapeDtypeStruct((GP, H), jnp.float32),
        ],
    )(xpad, w1pad, batch3)


def _bpass_body(u_ref, agg_ref, b1_ref, b3_ref, t_ref, s1_ref, s2_ref):
    i = pl.program_id(0)

    @pl.when(i == 0)
    def _():
        s1_ref[...] = jnp.zeros_like(s1_ref)
        s2_ref[...] = jnp.zeros_like(s2_ref)

    t = u_ref[...] + agg_ref[...] + b1_ref[...]
    t_ref[...] = t
    bg = b3_ref[0, 0, :]
    g0 = _wbase(bg)
    m = _onehot(bg, g0)
    s1_ref[pl.ds(g0, GW), :] += _segdot(m, t)
    s2_ref[pl.ds(g0, GW), :] += _segdot(m, t * t)


def _bpass(u, agg, b1, batch3):
    return pl.pallas_call(
        _bpass_body,
        grid=(NB,),
        in_specs=[
            pl.BlockSpec((B, H), lambda i: (i, 0)),
            pl.BlockSpec((B, H), lambda i: (i, 0)),
            pl.BlockSpec((1, H), lambda i: (0, 0)),
            pl.BlockSpec((1, 1, B), lambda i: (i, 0, 0)),
        ],
        out_specs=[
            pl.BlockSpec((B, H), lambda i: (i, 0)),
            pl.BlockSpec((GP, H), lambda i: (0, 0)),
            pl.BlockSpec((GP, H), lambda i: (0, 0)),
        ],
        out_shape=[
            jax.ShapeDtypeStruct((N, H), jnp.float32),
            jax.ShapeDtypeStruct((GP, H), jnp.float32),
            jax.ShapeDtypeStruct((GP, H), jnp.float32),
        ],
    )(u, agg, b1, batch3)


def _norm_coeffs(s1, s2, cnt, g, be, a):
    cntc = jnp.maximum(cnt, 1.0)
    mean = s1 / cntc
    var = s2 / cntc - (2.0 * a - a * a) * mean * mean
    var = jnp.maximum(var, 0.0)
    scale = g / jnp.sqrt(var + EPS)
    empty = cnt == 0.0
    scale = jnp.where(empty, 0.0, scale)
    shift = jnp.where(empty, 0.0, be - scale * a * mean)
    return scale, shift


def _dpass_body(t_ref, b3_ref, s1_ref, s2_ref, cnt_ref, g_ref, be_ref, a_ref,
                w2_ref, b2_ref, h_ref, o1_ref, o2_ref, scale_ref, shift_ref):
    i = pl.program_id(0)

    @pl.when(i == 0)
    def _():
        scale, shift = _norm_coeffs(s1_ref[...], s2_ref[...], cnt_ref[...],
                                    g_ref[...], be_ref[...], a_ref[...])
        scale_ref[...] = scale
        shift_ref[...] = shift
        o1_ref[...] = jnp.zeros_like(o1_ref)
        o2_ref[...] = jnp.zeros_like(o2_ref)

    bg = b3_ref[0, 0, :]
    g0 = _wbase(bg)
    m = _onehot(bg, g0)
    sc = jnp.dot(m, scale_ref[pl.ds(g0, GW), :], precision=lax.Precision.HIGHEST,
                 preferred_element_type=jnp.float32)
    sh = jnp.dot(m, shift_ref[pl.ds(g0, GW), :], precision=lax.Precision.HIGHEST,
                 preferred_element_type=jnp.float32)
    y = _leaky(sc * t_ref[...] + sh)
    h = jnp.dot(y, w2_ref[...], preferred_element_type=jnp.float32) + b2_ref[...]
    h_ref[...] = h
    o1_ref[pl.ds(g0, GW), :] += _segdot(m, h)
    o2_ref[pl.ds(g0, GW), :] += _segdot(m, h * h)


def _dpass(t, batch3, s1, s2, cnt, g, be, a, w2, b2):
    return pl.pallas_call(
        _dpass_body,
        grid=(NB,),
        in_specs=[
            pl.BlockSpec((B, H), lambda i: (i, 0)),
            pl.BlockSpec((1, 1, B), lambda i: (i, 0, 0)),
            pl.BlockSpec((GP, H), lambda i: (0, 0)),
            pl.BlockSpec((GP, H), lambda i: (0, 0)),
            pl.BlockSpec((GP, H), lambda i: (0, 0)),
            pl.BlockSpec((1, H), lambda i: (0, 0)),
            pl.BlockSpec((1, H), lambda i: (0, 0)),
            pl.BlockSpec((1, H), lambda i: (0, 0)),
            pl.BlockSpec((H, H), lambda i: (0, 0)),
            pl.BlockSpec((1, H), lambda i: (0, 0)),
        ],
        out_specs=[
            pl.BlockSpec((B, H), lambda i: (i, 0)),
            pl.BlockSpec((GP, H), lambda i: (0, 0)),
            pl.BlockSpec((GP, H), lambda i: (0, 0)),
        ],
        out_shape=[
            jax.ShapeDtypeStruct((N, H), jnp.float32),
            jax.ShapeDtypeStruct((GP, H), jnp.float32),
            jax.ShapeDtypeStruct((GP, H), jnp.float32),
        ],
        scratch_shapes=[
            pltpu.VMEM((GP, H), jnp.float32),
            pltpu.VMEM((GP, H), jnp.float32),
        ],
    )(t, batch3, s1, s2, cnt, g, be, a, w2, b2)


def _fpass_body(h_ref, b3_ref, s1_ref, s2_ref, cnt_ref, g_ref, be_ref, a_ref,
                w1_ref, u_ref, scale_ref, shift_ref):
    i = pl.program_id(0)

    @pl.when(i == 0)
    def _():
        scale, shift = _norm_coeffs(s1_ref[...], s2_ref[...], cnt_ref[...],
                                    g_ref[...], be_ref[...], a_ref[...])
        scale_ref[...] = scale
        shift_ref[...] = shift

    bg = b3_ref[0, 0, :]
    g0 = _wbase(bg)
    m = _onehot(bg, g0)
    sc = jnp.dot(m, scale_ref[pl.ds(g0, GW), :], precision=lax.Precision.HIGHEST,
                 preferred_element_type=jnp.float32)
    sh = jnp.dot(m, shift_ref[pl.ds(g0, GW), :], precision=lax.Precision.HIGHEST,
                 preferred_element_type=jnp.float32)
    hn = _leaky(sc * h_ref[...] + sh)
    u_ref[...] = jnp.dot(hn, w1_ref[...], preferred_element_type=jnp.float32)


def _fpass(h, batch3, s1, s2, cnt, g, be, a, w1n):
    return pl.pallas_call(
        _fpass_body,
        grid=(NB,),
        in_specs=[
            pl.BlockSpec((B, H), lambda i: (i, 0)),
            pl.BlockSpec((1, 1, B), lambda i: (i, 0, 0)),
            pl.BlockSpec((GP, H), lambda i: (0, 0)),
            pl.BlockSpec((GP, H), lambda i: (0, 0)),
            pl.BlockSpec((GP, H), lambda i: (0, 0)),
            pl.BlockSpec((1, H), lambda i: (0, 0)),
            pl.BlockSpec((1, H), lambda i: (0, 0)),
            pl.BlockSpec((1, H), lambda i: (0, 0)),
            pl.BlockSpec((H, H), lambda i: (0, 0)),
        ],
        out_specs=pl.BlockSpec((B, H), lambda i: (i, 0)),
        out_shape=jax.ShapeDtypeStruct((N, H), jnp.float32),
        scratch_shapes=[
            pltpu.VMEM((GP, H), jnp.float32),
            pltpu.VMEM((GP, H), jnp.float32),
        ],
    )(h, batch3, s1, s2, cnt, g, be, a, w1n)


def _head_body(s_ref, cnt_ref, mx_ref, w1_ref, b1_ref, w2_ref, b2_ref, o_ref):
    s = s_ref[...]
    cntc = jnp.maximum(cnt_ref[...], 1.0)
    z = jnp.concatenate([s / cntc, s, mx_ref[...]], axis=1)
    h1 = jnp.dot(z, w1_ref[...], preferred_element_type=jnp.float32) + b1_ref[...]
    h1 = _leaky(h1)
    o_ref[...] = jnp.dot(h1, w2_ref[...], preferred_element_type=jnp.float32) + b2_ref[...]


def _head(s, cnt, mx, fc1w, fc1b, fc2w8, fc2b8):
    return pl.pallas_call(
        _head_body,
        out_shape=jax.ShapeDtypeStruct((G, 8), jnp.float32),
    )(s, cnt, mx, fc1w, fc1b, fc2w8, fc2b8)


NPASS = 5
RB = 10000
RBP = 10112
TROWS = RBP // 16
ET = E // 16
CH = 2000
NVR = CH // 16
FCAP = 128


def _agg_body(u_hbm, src_hbm, dst_hbm, z_hbm, out_hbm,
              sels, seld, bsrc4, bdst4, rows4, dstbuf, srcbuf, accum,
              gsem, ssem):
    core = lax.axis_index("c")
    tid = lax.axis_index("s")
    lane = lax.broadcasted_iota(jnp.int32, (16,), 0)
    pad_src = (tid * 997 + lane * 61) % N

    def fire(pos, f, spare_row):
        del spare_row
        sl = f % 2

        @pl.when(f >= 2)
        def _():
            pltpu.make_async_copy(rows4.at[pl.ds(128 * sl, FCAP)],
                                  accum.at[bdst4.at[sl]], ssem.at[sl]).wait()

        for k in range(8):
            bsrc4[sl, pl.ds(16 * k, 16)] = sels[pl.ds(16 * k, 16)]
            bdst4[sl, pl.ds(16 * k, 16)] = seld[pl.ds(16 * k, 16)]
        pltpu.async_copy(u_hbm.at[bsrc4.at[sl]],
                         rows4.at[pl.ds(128 * sl, FCAP)], gsem.at[sl])

        @pl.when(f >= 1)
        def _():
            sp = (f - 1) % 2
            pltpu.make_async_copy(u_hbm.at[bsrc4.at[sp]],
                                  rows4.at[pl.ds(128 * sp, FCAP)],
                                  gsem.at[sp]).wait()
            pltpu.async_copy(rows4.at[pl.ds(128 * sp, FCAP)],
                             accum.at[bdst4.at[sp]], ssem.at[sp], add=True)

        tl_s = sels[pl.ds(FCAP, 16)]
        tl_d = seld[pl.ds(FCAP, 16)]
        sels[pl.ds(0, 16)] = tl_s
        seld[pl.ds(0, 16)] = tl_d
        return pos - FCAP, f + 1

    for p in range(NPASS):
        bkt = 2 * p + core
        lo = bkt * RB
        spare_row = RB + tid
        pltpu.sync_copy(z_hbm, accum.at[pl.ds(TROWS * tid, TROWS)])
        plsc.subcore_barrier()

        def vreg_step(v, carry, _lo=lo, _spare=spare_row):
            pos, f = carry
            d = dstbuf[pl.ds(16 * v, 16)]
            sv = srcbuf[pl.ds(16 * v, 16)]
            m = (d >= _lo) & (d < _lo + RB)
            dl = jnp.where(m, d - _lo, _spare)
            mi = m.astype(jnp.int32)
            excl = plsc.cumsum(mi) - mi
            idx = jnp.where(m, pos + excl, 2 * FCAP)
            plsc.store_scatter(sels, [idx], sv)
            plsc.store_scatter(seld, [idx], dl)
            pos = pos + jnp.sum(mi)
            return lax.cond(pos >= FCAP,
                            lambda q, g: fire(q, g, _spare),
                            lambda q, g: (q, g), pos, f)

        def chunk_step(c, carry, _vs=vreg_step):
            base = tid * ET + c * CH
            pltpu.sync_copy(dst_hbm.at[pl.ds(base, CH)], dstbuf)
            pltpu.sync_copy(src_hbm.at[pl.ds(base, CH)], srcbuf)
            return lax.fori_loop(0, NVR, _vs, carry)

        pos, f = lax.fori_loop(0, ET // CH, chunk_step, (0, 0))
        spare_v = jnp.full((16,), spare_row, jnp.int32)
        for k in range(8):
            sels[pl.ds(pos + 16 * k, 16)] = pad_src
            seld[pl.ds(pos + 16 * k, 16)] = spare_v
        _, f = fire(FCAP, f, spare_row)
        sl = (f - 1) % 2
        pltpu.make_async_copy(u_hbm.at[bsrc4.at[sl]],
                              rows4.at[pl.ds(128 * sl, FCAP)], gsem.at[sl]).wait()
        pltpu.async_copy(rows4.at[pl.ds(128 * sl, FCAP)],
                         accum.at[bdst4.at[sl]], ssem.at[sl], add=True)
        for dd in range(1, 3):
            @pl.when(f >= dd)
            def _(_d=dd):
                sd = (f - _d) % 2
                pltpu.make_async_copy(rows4.at[pl.ds(128 * sd, FCAP)],
                                      accum.at[bdst4.at[sd]], ssem.at[sd]).wait()
        plsc.subcore_barrier()
        r0 = TROWS * tid
        for k in range(8):
            pltpu.sync_copy(accum.at[pl.ds(r0 + 72 * k, 72)],
                            out_hbm.at[bkt, pl.ds(r0 + 72 * k, 72)])
        pltpu.sync_copy(accum.at[pl.ds(r0 + 576, 56)],
                        out_hbm.at[bkt, pl.ds(r0 + 576, 56)])
        plsc.subcore_barrier()


def _sc_agg(u, src, dst, zeros782):
    mesh = plsc.VectorSubcoreMesh(core_axis_name="c", subcore_axis_name="s")
    out = pl.kernel(
        _agg_body,
        out_type=jax.ShapeDtypeStruct((2 * NPASS, RBP, H), jnp.float32),
        mesh=mesh,
        compiler_params=pltpu.CompilerParams(needs_layout_passes=False),
        scratch_types=[
            pltpu.VMEM((272,), jnp.int32),
            pltpu.VMEM((272,), jnp.int32),
            pltpu.VMEM((2, FCAP), jnp.int32),
            pltpu.VMEM((2, FCAP), jnp.int32),
            pltpu.VMEM((2 * FCAP, H), jnp.float32),
            pltpu.VMEM((CH,), jnp.int32),
            pltpu.VMEM((CH,), jnp.int32),
            pltpu.VMEM_SHARED((RBP, H), jnp.float32),
            pltpu.SemaphoreType.DMA((2,)),
            pltpu.SemaphoreType.DMA((2,)),
        ],
    )(u, src, dst, zeros782)
    return out[:, :RB, :].reshape(N, H)


def _maxpool_body(h_hbm, cnt_hbm, out_hbm, cntv, offs, hv, stag, sem):
    core = lax.axis_index("c")
    tid = lax.axis_index("s")
    wid = core * 16 + tid
    lane = lax.broadcasted_iota(jnp.int32, (16,), 0)
    pltpu.sync_copy(cnt_hbm, cntv)
    def off_step(k, carry):
        v = cntv[pl.ds(16 * k, 16)]
        c = plsc.cumsum(v)
        offs[pl.ds(16 * k, 16)] = c - v + carry
        return carry + jnp.sum(v)

    total = lax.fori_loop(0, 16, off_step, jnp.int32(0))
    offs[pl.ds(256, 16)] = jnp.full((16,), total, jnp.int32)
    ov = offs[pl.ds(8 * wid, 16)]

    def pick(j):
        return jnp.sum(jnp.where(lane == j, ov, 0))

    RC = 48
    for gl in range(8):
        start = pick(gl)
        end = pick(gl + 1)
        s8 = (start // 8) * 8
        nch = (end - s8 + RC - 9) // (RC - 8) + 1

        def chunk(j, accs, _s=start, _e=end, _s8=s8):
            r0 = jnp.minimum(_s8 + (RC - 8) * j, N - RC)
            pltpu.async_copy(h_hbm.at[pl.ds(r0, RC)], hv, sem).wait()
            def row(r, accs2):
                valid = (r0 + r >= _s) & (r0 + r < _e)
                out = []
                for k in range(8):
                    x = hv[r, pl.ds(16 * k, 16)]
                    out.append(jnp.where(valid, jnp.maximum(accs2[k], x), accs2[k]))
                return tuple(out)
            return lax.fori_loop(0, RC, row, accs)

        neg = jnp.full((16,), -jnp.inf, jnp.float32)
        accs = lax.fori_loop(0, nch, chunk, (neg,) * 8)
        for k in range(8):
            stag[pl.ds(128 * gl + 16 * k, 16)] = accs[k]
    pltpu.sync_copy(stag, out_hbm.at[pl.ds(1024 * wid, 1024)])


def _sc_maxpool(h, cnt1d):
    mesh = plsc.VectorSubcoreMesh(core_axis_name="c", subcore_axis_name="s")
    out = pl.kernel(
        _maxpool_body,
        out_type=jax.ShapeDtypeStruct((G * H,), jnp.float32),
        mesh=mesh,
        compiler_params=pltpu.CompilerParams(needs_layout_passes=False),
        scratch_types=[
            pltpu.VMEM((G,), jnp.int32),
            pltpu.VMEM((272,), jnp.int32),
            pltpu.VMEM((48, H), jnp.float32),
            pltpu.VMEM((1024,), jnp.float32),
            pltpu.SemaphoreType.DMA,
        ],
    )(h, cnt1d)
    return out.reshape(G, H)


def kernel(x, edge_attr, params, edge_index, batch):
    del edge_attr
    src = edge_index[0].astype(jnp.int32)
    dst = edge_index[1].astype(jnp.int32)
    batch = batch.astype(jnp.int32)
    batch3 = batch.reshape(NB, 1, B)

    xpad = jnp.pad(x, ((0, 0), (0, H - x.shape[1])))
    lp = params['layers']
    w1pad = jnp.pad(lp[0]['W1'], ((0, H - lp[0]['W1'].shape[0]), (0, 0)))

    u, cnt = _p0(xpad, w1pad, batch3)
    cnt1d = cnt[:G, 0].astype(jnp.int32)
    zeros782 = jnp.zeros((TROWS, H), jnp.float32)

    row = lambda v: v.reshape(1, H)
    s1 = s2 = None
    for i in range(4):
        p = lp[i]
        agg = _sc_agg(u, src, dst, zeros782)
        t, s1, s2 = _bpass(u, agg, row(p['b1']), batch3)
        h, s1, s2 = _dpass(t, batch3, s1, s2, cnt, row(p['g1']), row(p['be1']),
                           row(p['a1']), p['W2'], row(p['b2']))
        if i < 3:
            n = params['norms'][i]
            u = _fpass(h, batch3, s1, s2, cnt, row(n['g']), row(n['be']),
                       row(n['a']), lp[i + 1]['W1'])

    mx = _sc_maxpool(h, cnt1d)
    out8 = _head(s1[:G], cnt[:G], mx, params['fc1_W'],
                 params['fc1_b'].reshape(1, 64),
                 jnp.pad(params['fc2_W'], ((0, 0), (0, 5))),
                 jnp.pad(params['fc2_b'], (0, 5)).reshape(1, 8))
    return out8[:, :3]


# speedup vs baseline: 1.3259x; 1.3259x over previous
import functools

import jax
import jax.numpy as jnp
from jax import lax
from jax.experimental import pallas as pl
from jax.experimental.pallas import tpu as pltpu
from jax.experimental.pallas import tpu_sc as plsc

N = 100000
E = 1600000
G = 256
H = 128
B = 1000
NB = N // B
GW = 24
GP = G + 16
EPS = 1e-5


def _onehot(bg, g0):
    return (bg[:, None] - g0 == lax.broadcasted_iota(jnp.int32, (B, GW), 1)).astype(jnp.float32)


def _wbase(bg):
    return (bg[0] // 8) * 8


def _segdot(m, t):
    return lax.dot_general(m, t, (((0,), (0,)), ((), ())),
                           precision=lax.Precision.HIGHEST,
                           preferred_element_type=jnp.float32)


def _leaky(x):
    return jnp.where(x >= 0, x, 0.01 * x)


def _p0_body(x_ref, w_ref, b3_ref, u_ref, cnt_ref):
    i = pl.program_id(0)

    @pl.when(i == 0)
    def _():
        cnt_ref[...] = jnp.zeros_like(cnt_ref)

    u_ref[...] = jnp.dot(x_ref[...], w_ref[...], preferred_element_type=jnp.float32)
    bg = b3_ref[0, 0, :]
    g0 = _wbase(bg)
    m = _onehot(bg, g0)
    cnt_ref[pl.ds(g0, GW), :] += _segdot(m, jnp.ones((B, H), jnp.float32))


def _p0(xpad, w1pad, batch3):
    return pl.pallas_call(
        _p0_body,
        grid=(NB,),
        in_specs=[
            pl.BlockSpec((B, H), lambda i: (i, 0)),
            pl.BlockSpec((H, H), lambda i: (0, 0)),
            pl.BlockSpec((1, 1, B), lambda i: (i, 0, 0)),
        ],
        out_specs=[
            pl.BlockSpec((B, H), lambda i: (i, 0)),
            pl.BlockSpec((GP, H), lambda i: (0, 0)),
        ],
        out_shape=[
            jax.ShapeDtypeStruct((N, H), jnp.float32),
            jax.ShapeDtypeStruct((GP, H), jnp.float32),
        ],
    )(xpad, w1pad, batch3)


def _bpass_body(u_ref, agg_ref, b1_ref, b3_ref, t_ref, s1_ref, s2_ref):
    i = pl.program_id(0)

    @pl.when(i == 0)
    def _():
        s1_ref[...] = jnp.zeros_like(s1_ref)
        s2_ref[...] = jnp.zeros_like(s2_ref)

    t = u_ref[...] + agg_ref[...] + b1_ref[...]
    t_ref[...] = t
    bg = b3_ref[0, 0, :]
    g0 = _wbase(bg)
    m = _onehot(bg, g0)
    s1_ref[pl.ds(g0, GW), :] += _segdot(m, t)
    s2_ref[pl.ds(g0, GW), :] += _segdot(m, t * t)


def _bpass(u, agg, b1, batch3):
    return pl.pallas_call(
        _bpass_body,
        grid=(NB,),
        in_specs=[
            pl.BlockSpec((B, H), lambda i: (i, 0)),
            pl.BlockSpec((B, H), lambda i: (i, 0)),
            pl.BlockSpec((1, H), lambda i: (0, 0)),
            pl.BlockSpec((1, 1, B), lambda i: (i, 0, 0)),
        ],
        out_specs=[
            pl.BlockSpec((B, H), lambda i: (i, 0)),
            pl.BlockSpec((GP, H), lambda i: (0, 0)),
            pl.BlockSpec((GP, H), lambda i: (0, 0)),
        ],
        out_shape=[
            jax.ShapeDtypeStruct((N, H), jnp.float32),
            jax.ShapeDtypeStruct((GP, H), jnp.float32),
            jax.ShapeDtypeStruct((GP, H), jnp.float32),
        ],
    )(u, agg, b1, batch3)


def _norm_coeffs(s1, s2, cnt, g, be, a):
    cntc = jnp.maximum(cnt, 1.0)
    mean = s1 / cntc
    var = s2 / cntc - (2.0 * a - a * a) * mean * mean
    var = jnp.maximum(var, 0.0)
    scale = g / jnp.sqrt(var + EPS)
    empty = cnt == 0.0
    scale = jnp.where(empty, 0.0, scale)
    shift = jnp.where(empty, 0.0, be - scale * a * mean)
    return scale, shift


def _dpass_body(t_ref, b3_ref, s1_ref, s2_ref, cnt_ref, g_ref, be_ref, a_ref,
                w2_ref, b2_ref, h_ref, o1_ref, o2_ref, scale_ref, shift_ref):
    i = pl.program_id(0)

    @pl.when(i == 0)
    def _():
        scale, shift = _norm_coeffs(s1_ref[...], s2_ref[...], cnt_ref[...],
                                    g_ref[...], be_ref[...], a_ref[...])
        scale_ref[...] = scale
        shift_ref[...] = shift
        o1_ref[...] = jnp.zeros_like(o1_ref)
        o2_ref[...] = jnp.zeros_like(o2_ref)

    bg = b3_ref[0, 0, :]
    g0 = _wbase(bg)
    m = _onehot(bg, g0)
    sc = jnp.dot(m, scale_ref[pl.ds(g0, GW), :], precision=lax.Precision.HIGHEST,
                 preferred_element_type=jnp.float32)
    sh = jnp.dot(m, shift_ref[pl.ds(g0, GW), :], precision=lax.Precision.HIGHEST,
                 preferred_element_type=jnp.float32)
    y = _leaky(sc * t_ref[...] + sh)
    h = jnp.dot(y, w2_ref[...], preferred_element_type=jnp.float32) + b2_ref[...]
    h_ref[...] = h
    o1_ref[pl.ds(g0, GW), :] += _segdot(m, h)
    o2_ref[pl.ds(g0, GW), :] += _segdot(m, h * h)


def _dpass(t, batch3, s1, s2, cnt, g, be, a, w2, b2):
    return pl.pallas_call(
        _dpass_body,
        grid=(NB,),
        in_specs=[
            pl.BlockSpec((B, H), lambda i: (i, 0)),
            pl.BlockSpec((1, 1, B), lambda i: (i, 0, 0)),
            pl.BlockSpec((GP, H), lambda i: (0, 0)),
            pl.BlockSpec((GP, H), lambda i: (0, 0)),
            pl.BlockSpec((GP, H), lambda i: (0, 0)),
            pl.BlockSpec((1, H), lambda i: (0, 0)),
            pl.BlockSpec((1, H), lambda i: (0, 0)),
            pl.BlockSpec((1, H), lambda i: (0, 0)),
            pl.BlockSpec((H, H), lambda i: (0, 0)),
            pl.BlockSpec((1, H), lambda i: (0, 0)),
        ],
        out_specs=[
            pl.BlockSpec((B, H), lambda i: (i, 0)),
            pl.BlockSpec((GP, H), lambda i: (0, 0)),
            pl.BlockSpec((GP, H), lambda i: (0, 0)),
        ],
        out_shape=[
            jax.ShapeDtypeStruct((N, H), jnp.float32),
            jax.ShapeDtypeStruct((GP, H), jnp.float32),
            jax.ShapeDtypeStruct((GP, H), jnp.float32),
        ],
        scratch_shapes=[
            pltpu.VMEM((GP, H), jnp.float32),
            pltpu.VMEM((GP, H), jnp.float32),
        ],
    )(t, batch3, s1, s2, cnt, g, be, a, w2, b2)


def _fpass_body(h_ref, b3_ref, s1_ref, s2_ref, cnt_ref, g_ref, be_ref, a_ref,
                w1_ref, u_ref, scale_ref, shift_ref):
    i = pl.program_id(0)

    @pl.when(i == 0)
    def _():
        scale, shift = _norm_coeffs(s1_ref[...], s2_ref[...], cnt_ref[...],
                                    g_ref[...], be_ref[...], a_ref[...])
        scale_ref[...] = scale
        shift_ref[...] = shift

    bg = b3_ref[0, 0, :]
    g0 = _wbase(bg)
    m = _onehot(bg, g0)
    sc = jnp.dot(m, scale_ref[pl.ds(g0, GW), :], precision=lax.Precision.HIGHEST,
                 preferred_element_type=jnp.float32)
    sh = jnp.dot(m, shift_ref[pl.ds(g0, GW), :], precision=lax.Precision.HIGHEST,
                 preferred_element_type=jnp.float32)
    hn = _leaky(sc * h_ref[...] + sh)
    u_ref[...] = jnp.dot(hn, w1_ref[...], preferred_element_type=jnp.float32)


def _fpass(h, batch3, s1, s2, cnt, g, be, a, w1n):
    return pl.pallas_call(
        _fpass_body,
        grid=(NB,),
        in_specs=[
            pl.BlockSpec((B, H), lambda i: (i, 0)),
            pl.BlockSpec((1, 1, B), lambda i: (i, 0, 0)),
            pl.BlockSpec((GP, H), lambda i: (0, 0)),
            pl.BlockSpec((GP, H), lambda i: (0, 0)),
            pl.BlockSpec((GP, H), lambda i: (0, 0)),
            pl.BlockSpec((1, H), lambda i: (0, 0)),
            pl.BlockSpec((1, H), lambda i: (0, 0)),
            pl.BlockSpec((1, H), lambda i: (0, 0)),
            pl.BlockSpec((H, H), lambda i: (0, 0)),
        ],
        out_specs=pl.BlockSpec((B, H), lambda i: (i, 0)),
        out_shape=jax.ShapeDtypeStruct((N, H), jnp.float32),
        scratch_shapes=[
            pltpu.VMEM((GP, H), jnp.float32),
            pltpu.VMEM((GP, H), jnp.float32),
        ],
    )(h, batch3, s1, s2, cnt, g, be, a, w1n)


def _head_body(s_ref, cnt_ref, mx_ref, w1_ref, b1_ref, w2_ref, b2_ref, o_ref):
    s = s_ref[...]
    cntc = jnp.maximum(cnt_ref[...], 1.0)
    z = jnp.concatenate([s / cntc, s, mx_ref[...]], axis=1)
    h1 = jnp.dot(z, w1_ref[...], preferred_element_type=jnp.float32) + b1_ref[...]
    h1 = _leaky(h1)
    o_ref[...] = jnp.dot(h1, w2_ref[...], preferred_element_type=jnp.float32) + b2_ref[...]


def _head(s, cnt, mx, fc1w, fc1b, fc2w8, fc2b8):
    return pl.pallas_call(
        _head_body,
        out_shape=jax.ShapeDtypeStruct((G, 8), jnp.float32),
    )(s, cnt, mx, fc1w, fc1b, fc2w8, fc2b8)


NPASS = 4
RB = 12500
RBP = 12544
TROWS = RBP // 16
ET = E // 16
CH = 2000
NVR = CH // 16
FCAP = 128


def _agg_body(u_hbm, src_hbm, dst_hbm, z_hbm, out_hbm,
              sels, seld, bsrc, bdst, rows, dstbuf, srcbuf, accum, gsem, ssem):
    core = lax.axis_index("c")
    tid = lax.axis_index("s")
    lane = lax.broadcasted_iota(jnp.int32, (16,), 0)
    pad_src = (tid * 997 + lane * 61) % N

    def fire(pos):
        for k in range(8):
            bsrc[pl.ds(16 * k, 16)] = sels[pl.ds(16 * k, 16)]
            bdst[pl.ds(16 * k, 16)] = seld[pl.ds(16 * k, 16)]
        pltpu.async_copy(u_hbm.at[bsrc], rows, gsem).wait()
        pltpu.async_copy(rows, accum.at[bdst], ssem, add=True).wait()
        tl_s = sels[pl.ds(FCAP, 16)]
        tl_d = seld[pl.ds(FCAP, 16)]
        sels[pl.ds(0, 16)] = tl_s
        seld[pl.ds(0, 16)] = tl_d
        return pos - FCAP

    for p in range(NPASS):
        bkt = 2 * p + core
        lo = bkt * RB
        spare_row = RB + tid
        pltpu.sync_copy(z_hbm, accum.at[pl.ds(TROWS * tid, TROWS)])
        plsc.subcore_barrier()

        def vreg_step(v, pos, _lo=lo, _spare=spare_row):
            d = dstbuf[pl.ds(16 * v, 16)]
            sv = srcbuf[pl.ds(16 * v, 16)]
            m = (d >= _lo) & (d < _lo + RB)
            dl = jnp.where(m, d - _lo, _spare)
            mi = m.astype(jnp.int32)
            excl = plsc.cumsum(mi) - mi
            idx = jnp.where(m, pos + excl, 2 * FCAP)
            plsc.store_scatter(sels, [idx], sv)
            plsc.store_scatter(seld, [idx], dl)
            pos = pos + jnp.sum(mi)
            return lax.cond(pos >= FCAP, fire, lambda q: q, pos)

        def chunk_step(c, pos, _vs=vreg_step):
            base = tid * ET + c * CH
            pltpu.sync_copy(dst_hbm.at[pl.ds(base, CH)], dstbuf)
            pltpu.sync_copy(src_hbm.at[pl.ds(base, CH)], srcbuf)
            return lax.fori_loop(0, NVR, _vs, pos)

        pos = lax.fori_loop(0, ET // CH, chunk_step, 0)
        spare_v = jnp.full((16,), spare_row, jnp.int32)
        for k in range(8):
            sels[pl.ds(pos + 16 * k, 16)] = pad_src
            seld[pl.ds(pos + 16 * k, 16)] = spare_v
        fire(FCAP)
        plsc.subcore_barrier()
        r0 = TROWS * tid
        for k in range(14):
            pltpu.sync_copy(accum.at[pl.ds(r0 + 56 * k, 56)],
                            out_hbm.at[bkt, pl.ds(r0 + 56 * k, 56)])
        plsc.subcore_barrier()


def _sc_agg(u, src, dst, zeros782):
    mesh = plsc.VectorSubcoreMesh(core_axis_name="c", subcore_axis_name="s")
    out = pl.kernel(
        _agg_body,
        out_type=jax.ShapeDtypeStruct((8, RBP, H), jnp.float32),
        mesh=mesh,
        compiler_params=pltpu.CompilerParams(needs_layout_passes=False),
        scratch_types=[
            pltpu.VMEM((272,), jnp.int32),
            pltpu.VMEM((272,), jnp.int32),
            pltpu.VMEM((FCAP,), jnp.int32),
            pltpu.VMEM((FCAP,), jnp.int32),
            pltpu.VMEM((FCAP, H), jnp.float32),
            pltpu.VMEM((CH,), jnp.int32),
            pltpu.VMEM((CH,), jnp.int32),
            pltpu.VMEM_SHARED((RBP, H), jnp.float32),
            pltpu.SemaphoreType.DMA,
            pltpu.SemaphoreType.DMA,
        ],
    )(u, src, dst, zeros782)
    return out[:, :RB, :].reshape(N, H)


def _maxpool_body(h_hbm, cnt_hbm, out_hbm, cntv, offs, hv, stag, sem):
    core = lax.axis_index("c")
    tid = lax.axis_index("s")
    wid = core * 16 + tid
    lane = lax.broadcasted_iota(jnp.int32, (16,), 0)
    pltpu.sync_copy(cnt_hbm, cntv)
    def off_step(k, carry):
        v = cntv[pl.ds(16 * k, 16)]
        c = plsc.cumsum(v)
        offs[pl.ds(16 * k, 16)] = c - v + carry
        return carry + jnp.sum(v)

    total = lax.fori_loop(0, 16, off_step, jnp.int32(0))
    offs[pl.ds(256, 16)] = jnp.full((16,), total, jnp.int32)
    ov = offs[pl.ds(8 * wid, 16)]

    def pick(j):
        return jnp.sum(jnp.where(lane == j, ov, 0))

    RC = 48
    for gl in range(8):
        start = pick(gl)
        end = pick(gl + 1)
        s8 = (start // 8) * 8
        nch = (end - s8 + RC - 9) // (RC - 8) + 1

        def chunk(j, accs, _s=start, _e=end, _s8=s8):
            r0 = jnp.minimum(_s8 + (RC - 8) * j, N - RC)
            pltpu.async_copy(h_hbm.at[pl.ds(r0, RC)], hv, sem).wait()
            def row(r, accs2):
                valid = (r0 + r >= _s) & (r0 + r < _e)
                out = []
                for k in range(8):
                    x = hv[r, pl.ds(16 * k, 16)]
                    out.append(jnp.where(valid, jnp.maximum(accs2[k], x), accs2[k]))
                return tuple(out)
            return lax.fori_loop(0, RC, row, accs)

        neg = jnp.full((16,), -jnp.inf, jnp.float32)
        accs = lax.fori_loop(0, nch, chunk, (neg,) * 8)
        for k in range(8):
            stag[pl.ds(128 * gl + 16 * k, 16)] = accs[k]
    pltpu.sync_copy(stag, out_hbm.at[pl.ds(1024 * wid, 1024)])


def _sc_maxpool(h, cnt1d):
    mesh = plsc.VectorSubcoreMesh(core_axis_name="c", subcore_axis_name="s")
    out = pl.kernel(
        _maxpool_body,
        out_type=jax.ShapeDtypeStruct((G * H,), jnp.float32),
        mesh=mesh,
        compiler_params=pltpu.CompilerParams(needs_layout_passes=False),
        scratch_types=[
            pltpu.VMEM((G,), jnp.int32),
            pltpu.VMEM((272,), jnp.int32),
            pltpu.VMEM((48, H), jnp.float32),
            pltpu.VMEM((1024,), jnp.float32),
            pltpu.SemaphoreType.DMA,
        ],
    )(h, cnt1d)
    return out.reshape(G, H)


def kernel(x, edge_attr, params, edge_index, batch):
    del edge_attr
    src = edge_index[0].astype(jnp.int32)
    dst = edge_index[1].astype(jnp.int32)
    batch = batch.astype(jnp.int32)
    batch3 = batch.reshape(NB, 1, B)

    xpad = jnp.pad(x, ((0, 0), (0, H - x.shape[1])))
    lp = params['layers']
    w1pad = jnp.pad(lp[0]['W1'], ((0, H - lp[0]['W1'].shape[0]), (0, 0)))

    u, cnt = _p0(xpad, w1pad, batch3)
    cnt1d = cnt[:G, 0].astype(jnp.int32)
    zeros782 = jnp.zeros((TROWS, H), jnp.float32)

    row = lambda v: v.reshape(1, H)
    s1 = s2 = None
    for i in range(4):
        p = lp[i]
        agg = _sc_agg(u, src, dst, zeros782)
        t, s1, s2 = _bpass(u, agg, row(p['b1']), batch3)
        h, s1, s2 = _dpass(t, batch3, s1, s2, cnt, row(p['g1']), row(p['be1']),
                           row(p['a1']), p['W2'], row(p['b2']))
        if i < 3:
            n = params['norms'][i]
            u = _fpass(h, batch3, s1, s2, cnt, row(n['g']), row(n['be']),
                       row(n['a']), lp[i + 1]['W1'])

    mx = _sc_maxpool(h, cnt1d)
    out8 = _head(s1[:G], cnt[:G], mx, params['fc1_W'],
                 params['fc1_b'].reshape(1, 64),
                 jnp.pad(params['fc2_W'], ((0, 0), (0, 5))),
                 jnp.pad(params['fc2_b'], (0, 5)).reshape(1, 8))
    return out8[:, :3]


# speedup vs baseline: 1.5884x; 1.1980x over previous
import functools

import jax
import jax.numpy as jnp
from jax import lax
from jax.experimental import pallas as pl
from jax.experimental.pallas import tpu as pltpu
from jax.experimental.pallas import tpu_sc as plsc

N = 100000
E = 1600000
G = 256
H = 128
B = 1000
NB = N // B
GW = 24
GP = G + 16
EPS = 1e-5


def _onehot(bg, g0):
    return (bg[:, None] - g0 == lax.broadcasted_iota(jnp.int32, (B, GW), 1)).astype(jnp.float32)


def _wbase(bg):
    return (bg[0] // 8) * 8


def _segdot(m, t):
    return lax.dot_general(m, t, (((0,), (0,)), ((), ())),
                           precision=lax.Precision.HIGHEST,
                           preferred_element_type=jnp.float32)


def _leaky(x):
    return jnp.where(x >= 0, x, 0.01 * x)


def _p0_body(x_ref, w_ref, b3_ref, u_ref, cnt_ref):
    i = pl.program_id(0)

    @pl.when(i == 0)
    def _():
        cnt_ref[...] = jnp.zeros_like(cnt_ref)

    u_ref[...] = jnp.dot(x_ref[...], w_ref[...], preferred_element_type=jnp.float32)
    bg = b3_ref[0, 0, :]
    g0 = _wbase(bg)
    m = _onehot(bg, g0)
    cnt_ref[pl.ds(g0, GW), :] += _segdot(m, jnp.ones((B, H), jnp.float32))


def _p0(xpad, w1pad, batch3):
    return pl.pallas_call(
        _p0_body,
        grid=(NB,),
        in_specs=[
            pl.BlockSpec((B, H), lambda i: (i, 0)),
            pl.BlockSpec((H, H), lambda i: (0, 0)),
            pl.BlockSpec((1, 1, B), lambda i: (i, 0, 0)),
        ],
        out_specs=[
            pl.BlockSpec((B, H), lambda i: (i, 0)),
            pl.BlockSpec((GP, H), lambda i: (0, 0)),
        ],
        out_shape=[
            jax.ShapeDtypeStruct((N, H), jnp.float32),
            jax.ShapeDtypeStruct((GP, H), jnp.float32),
        ],
    )(xpad, w1pad, batch3)


def _bpass_body(u_ref, agg_ref, b1_ref, b3_ref, t_ref, s1_ref, s2_ref):
    i = pl.program_id(0)

    @pl.when(i == 0)
    def _():
        s1_ref[...] = jnp.zeros_like(s1_ref)
        s2_ref[...] = jnp.zeros_like(s2_ref)

    t = u_ref[...] + agg_ref[...] + b1_ref[...]
    t_ref[...] = t
    bg = b3_ref[0, 0, :]
    g0 = _wbase(bg)
    m = _onehot(bg, g0)
    s1_ref[pl.ds(g0, GW), :] += _segdot(m, t)
    s2_ref[pl.ds(g0, GW), :] += _segdot(m, t * t)


def _bpass(u, agg, b1, batch3):
    return pl.pallas_call(
        _bpass_body,
        grid=(NB,),
        in_specs=[
            pl.BlockSpec((B, H), lambda i: (i, 0)),
            pl.BlockSpec((B, H), lambda i: (i, 0)),
            pl.BlockSpec((1, H), lambda i: (0, 0)),
            pl.BlockSpec((1, 1, B), lambda i: (i, 0, 0)),
        ],
        out_specs=[
            pl.BlockSpec((B, H), lambda i: (i, 0)),
            pl.BlockSpec((GP, H), lambda i: (0, 0)),
            pl.BlockSpec((GP, H), lambda i: (0, 0)),
        ],
        out_shape=[
            jax.ShapeDtypeStruct((N, H), jnp.float32),
            jax.ShapeDtypeStruct((GP, H), jnp.float32),
            jax.ShapeDtypeStruct((GP, H), jnp.float32),
        ],
    )(u, agg, b1, batch3)


def _norm_coeffs(s1, s2, cnt, g, be, a):
    cntc = jnp.maximum(cnt, 1.0)
    mean = s1 / cntc
    var = s2 / cntc - (2.0 * a - a * a) * mean * mean
    var = jnp.maximum(var, 0.0)
    scale = g / jnp.sqrt(var + EPS)
    empty = cnt == 0.0
    scale = jnp.where(empty, 0.0, scale)
    shift = jnp.where(empty, 0.0, be - scale * a * mean)
    return scale, shift


def _dpass_body(t_ref, b3_ref, s1_ref, s2_ref, cnt_ref, g_ref, be_ref, a_ref,
                w2_ref, b2_ref, h_ref, o1_ref, o2_ref, scale_ref, shift_ref):
    i = pl.program_id(0)

    @pl.when(i == 0)
    def _():
        scale, shift = _norm_coeffs(s1_ref[...], s2_ref[...], cnt_ref[...],
                                    g_ref[...], be_ref[...], a_ref[...])
        scale_ref[...] = scale
        shift_ref[...] = shift
        o1_ref[...] = jnp.zeros_like(o1_ref)
        o2_ref[...] = jnp.zeros_like(o2_ref)

    bg = b3_ref[0, 0, :]
    g0 = _wbase(bg)
    m = _onehot(bg, g0)
    sc = jnp.dot(m, scale_ref[pl.ds(g0, GW), :], precision=lax.Precision.HIGHEST,
                 preferred_element_type=jnp.float32)
    sh = jnp.dot(m, shift_ref[pl.ds(g0, GW), :], precision=lax.Precision.HIGHEST,
                 preferred_element_type=jnp.float32)
    y = _leaky(sc * t_ref[...] + sh)
    h = jnp.dot(y, w2_ref[...], preferred_element_type=jnp.float32) + b2_ref[...]
    h_ref[...] = h
    o1_ref[pl.ds(g0, GW), :] += _segdot(m, h)
    o2_ref[pl.ds(g0, GW), :] += _segdot(m, h * h)


def _dpass(t, batch3, s1, s2, cnt, g, be, a, w2, b2):
    return pl.pallas_call(
        _dpass_body,
        grid=(NB,),
        in_specs=[
            pl.BlockSpec((B, H), lambda i: (i, 0)),
            pl.BlockSpec((1, 1, B), lambda i: (i, 0, 0)),
            pl.BlockSpec((GP, H), lambda i: (0, 0)),
            pl.BlockSpec((GP, H), lambda i: (0, 0)),
            pl.BlockSpec((GP, H), lambda i: (0, 0)),
            pl.BlockSpec((1, H), lambda i: (0, 0)),
            pl.BlockSpec((1, H), lambda i: (0, 0)),
            pl.BlockSpec((1, H), lambda i: (0, 0)),
            pl.BlockSpec((H, H), lambda i: (0, 0)),
            pl.BlockSpec((1, H), lambda i: (0, 0)),
        ],
        out_specs=[
            pl.BlockSpec((B, H), lambda i: (i, 0)),
            pl.BlockSpec((GP, H), lambda i: (0, 0)),
            pl.BlockSpec((GP, H), lambda i: (0, 0)),
        ],
        out_shape=[
            jax.ShapeDtypeStruct((N, H), jnp.float32),
            jax.ShapeDtypeStruct((GP, H), jnp.float32),
            jax.ShapeDtypeStruct((GP, H), jnp.float32),
        ],
        scratch_shapes=[
            pltpu.VMEM((GP, H), jnp.float32),
            pltpu.VMEM((GP, H), jnp.float32),
        ],
    )(t, batch3, s1, s2, cnt, g, be, a, w2, b2)


def _fpass_body(h_ref, b3_ref, s1_ref, s2_ref, cnt_ref, g_ref, be_ref, a_ref,
                w1_ref, u_ref, scale_ref, shift_ref):
    i = pl.program_id(0)

    @pl.when(i == 0)
    def _():
        scale, shift = _norm_coeffs(s1_ref[...], s2_ref[...], cnt_ref[...],
                                    g_ref[...], be_ref[...], a_ref[...])
        scale_ref[...] = scale
        shift_ref[...] = shift

    bg = b3_ref[0, 0, :]
    g0 = _wbase(bg)
    m = _onehot(bg, g0)
    sc = jnp.dot(m, scale_ref[pl.ds(g0, GW), :], precision=lax.Precision.HIGHEST,
                 preferred_element_type=jnp.float32)
    sh = jnp.dot(m, shift_ref[pl.ds(g0, GW), :], precision=lax.Precision.HIGHEST,
                 preferred_element_type=jnp.float32)
    hn = _leaky(sc * h_ref[...] + sh)
    u_ref[...] = jnp.dot(hn, w1_ref[...], preferred_element_type=jnp.float32)


def _fpass(h, batch3, s1, s2, cnt, g, be, a, w1n):
    return pl.pallas_call(
        _fpass_body,
        grid=(NB,),
        in_specs=[
            pl.BlockSpec((B, H), lambda i: (i, 0)),
            pl.BlockSpec((1, 1, B), lambda i: (i, 0, 0)),
            pl.BlockSpec((GP, H), lambda i: (0, 0)),
            pl.BlockSpec((GP, H), lambda i: (0, 0)),
            pl.BlockSpec((GP, H), lambda i: (0, 0)),
            pl.BlockSpec((1, H), lambda i: (0, 0)),
            pl.BlockSpec((1, H), lambda i: (0, 0)),
            pl.BlockSpec((1, H), lambda i: (0, 0)),
            pl.BlockSpec((H, H), lambda i: (0, 0)),
        ],
        out_specs=pl.BlockSpec((B, H), lambda i: (i, 0)),
        out_shape=jax.ShapeDtypeStruct((N, H), jnp.float32),
        scratch_shapes=[
            pltpu.VMEM((GP, H), jnp.float32),
            pltpu.VMEM((GP, H), jnp.float32),
        ],
    )(h, batch3, s1, s2, cnt, g, be, a, w1n)


def _head_body(s_ref, cnt_ref, mx_ref, w1_ref, b1_ref, w2_ref, b2_ref, o_ref):
    s = s_ref[...]
    cntc = jnp.maximum(cnt_ref[...], 1.0)
    z = jnp.concatenate([s / cntc, s, mx_ref[...]], axis=1)
    h1 = jnp.dot(z, w1_ref[...], preferred_element_type=jnp.float32) + b1_ref[...]
    h1 = _leaky(h1)
    o_ref[...] = jnp.dot(h1, w2_ref[...], preferred_element_type=jnp.float32) + b2_ref[...]


def _head(s, cnt, mx, fc1w, fc1b, fc2w8, fc2b8):
    return pl.pallas_call(
        _head_body,
        out_shape=jax.ShapeDtypeStruct((G, 8), jnp.float32),
    )(s, cnt, mx, fc1w, fc1b, fc2w8, fc2b8)


NPASS = 4
RB = 12500
RBP = 12544
TROWS = RBP // 16
ET = E // 16
CH = 2000
NVR = CH // 16
FCAP = 96


def _agg_body(u_hbm, src_hbm, dst_hbm, z_hbm, out_hbm,
              sels, seld, bsrcA, bdstA, bsrcB, bdstB, rowsA, rowsB,
              dstbuf, srcbuf, accum, gsem, ssemA, ssemB):
    core = lax.axis_index("c")
    tid = lax.axis_index("s")
    lane = lax.broadcasted_iota(jnp.int32, (16,), 0)
    pad_src = (tid * 997 + lane * 61) % N

    def fire(pos, f):
        even = lax.rem(f, 2) == 0

        @pl.when(even)
        def _():
            @pl.when(f >= 2)
            def _():
                pltpu.make_async_copy(rowsA, accum.at[bdstA], ssemA).wait()
            for k in range(FCAP // 16):
                bsrcA[pl.ds(16 * k, 16)] = sels[pl.ds(16 * k, 16)]
                bdstA[pl.ds(16 * k, 16)] = seld[pl.ds(16 * k, 16)]
            pltpu.async_copy(u_hbm.at[bsrcA], rowsA, gsem).wait()
            pltpu.async_copy(rowsA, accum.at[bdstA], ssemA, add=True)

        @pl.when(jnp.logical_not(even))
        def _():
            @pl.when(f >= 2)
            def _():
                pltpu.make_async_copy(rowsB, accum.at[bdstB], ssemB).wait()
            for k in range(FCAP // 16):
                bsrcB[pl.ds(16 * k, 16)] = sels[pl.ds(16 * k, 16)]
                bdstB[pl.ds(16 * k, 16)] = seld[pl.ds(16 * k, 16)]
            pltpu.async_copy(u_hbm.at[bsrcB], rowsB, gsem).wait()
            pltpu.async_copy(rowsB, accum.at[bdstB], ssemB, add=True)

        tl_s = sels[pl.ds(FCAP, 16)]
        tl_d = seld[pl.ds(FCAP, 16)]
        sels[pl.ds(0, 16)] = tl_s
        seld[pl.ds(0, 16)] = tl_d
        return pos - FCAP, f + 1

    for p in range(NPASS):
        bkt = 2 * p + core
        lo = bkt * RB
        spare_row = RB + tid
        pltpu.sync_copy(z_hbm, accum.at[pl.ds(TROWS * tid, TROWS)])
        plsc.subcore_barrier()

        def vreg_step(v, carry, _lo=lo, _spare=spare_row):
            pos, f = carry
            d = dstbuf[pl.ds(16 * v, 16)]
            sv = srcbuf[pl.ds(16 * v, 16)]
            m = (d >= _lo) & (d < _lo + RB)
            dl = jnp.where(m, d - _lo, _spare)
            mi = m.astype(jnp.int32)
            excl = plsc.cumsum(mi) - mi
            idx = jnp.where(m, pos + excl, 2 * FCAP)
            plsc.store_scatter(sels, [idx], sv)
            plsc.store_scatter(seld, [idx], dl)
            pos = pos + jnp.sum(mi)
            return lax.cond(pos >= FCAP, fire, lambda q, g: (q, g), pos, f)

        def chunk_step(c, carry, _vs=vreg_step):
            base = tid * ET + c * CH
            pltpu.sync_copy(dst_hbm.at[pl.ds(base, CH)], dstbuf)
            pltpu.sync_copy(src_hbm.at[pl.ds(base, CH)], srcbuf)
            return lax.fori_loop(0, NVR, _vs, carry)

        pos, f = lax.fori_loop(0, ET // CH, chunk_step, (0, 0))
        spare_v = jnp.full((16,), spare_row, jnp.int32)
        for k in range(FCAP // 16):
            sels[pl.ds(pos + 16 * k, 16)] = pad_src
            seld[pl.ds(pos + 16 * k, 16)] = spare_v
        _, f = fire(FCAP, f)
        @pl.when(lax.rem(f - 1, 2) == 0)
        def _():
            pltpu.make_async_copy(rowsA, accum.at[bdstA], ssemA).wait()
            @pl.when(f >= 2)
            def _():
                pltpu.make_async_copy(rowsB, accum.at[bdstB], ssemB).wait()
        @pl.when(lax.rem(f - 1, 2) == 1)
        def _():
            pltpu.make_async_copy(rowsB, accum.at[bdstB], ssemB).wait()
            @pl.when(f >= 2)
            def _():
                pltpu.make_async_copy(rowsA, accum.at[bdstA], ssemA).wait()
        plsc.subcore_barrier()
        r0 = TROWS * tid
        for k in range(14):
            pltpu.sync_copy(accum.at[pl.ds(r0 + 56 * k, 56)],
                            out_hbm.at[bkt, pl.ds(r0 + 56 * k, 56)])
        plsc.subcore_barrier()


def _sc_agg(u, src, dst, zeros782):
    mesh = plsc.VectorSubcoreMesh(core_axis_name="c", subcore_axis_name="s")
    out = pl.kernel(
        _agg_body,
        out_type=jax.ShapeDtypeStruct((8, RBP, H), jnp.float32),
        mesh=mesh,
        compiler_params=pltpu.CompilerParams(needs_layout_passes=False),
        scratch_types=[
            pltpu.VMEM((272,), jnp.int32),
            pltpu.VMEM((272,), jnp.int32),
            pltpu.VMEM((FCAP,), jnp.int32),
            pltpu.VMEM((FCAP,), jnp.int32),
            pltpu.VMEM((FCAP,), jnp.int32),
            pltpu.VMEM((FCAP,), jnp.int32),
            pltpu.VMEM((FCAP, H), jnp.float32),
            pltpu.VMEM((FCAP, H), jnp.float32),
            pltpu.VMEM((CH,), jnp.int32),
            pltpu.VMEM((CH,), jnp.int32),
            pltpu.VMEM_SHARED((RBP, H), jnp.float32),
            pltpu.SemaphoreType.DMA,
            pltpu.SemaphoreType.DMA,
            pltpu.SemaphoreType.DMA,
        ],
    )(u, src, dst, zeros782)
    return out[:, :RB, :].reshape(N, H)


def _maxpool_body(h_hbm, cnt_hbm, out_hbm, cntv, offs, hv, stag, sem):
    core = lax.axis_index("c")
    tid = lax.axis_index("s")
    wid = core * 16 + tid
    lane = lax.broadcasted_iota(jnp.int32, (16,), 0)
    pltpu.sync_copy(cnt_hbm, cntv)
    def off_step(k, carry):
        v = cntv[pl.ds(16 * k, 16)]
        c = plsc.cumsum(v)
        offs[pl.ds(16 * k, 16)] = c - v + carry
        return carry + jnp.sum(v)

    total = lax.fori_loop(0, 16, off_step, jnp.int32(0))
    offs[pl.ds(256, 16)] = jnp.full((16,), total, jnp.int32)
    ov = offs[pl.ds(8 * wid, 16)]

    def pick(j):
        return jnp.sum(jnp.where(lane == j, ov, 0))

    RC = 48
    for gl in range(8):
        start = pick(gl)
        end = pick(gl + 1)
        s8 = (start // 8) * 8
        nch = (end - s8 + RC - 9) // (RC - 8) + 1

        def chunk(j, accs, _s=start, _e=end, _s8=s8):
            r0 = jnp.minimum(_s8 + (RC - 8) * j, N - RC)
            pltpu.async_copy(h_hbm.at[pl.ds(r0, RC)], hv, sem).wait()
            def row(r, accs2):
                valid = (r0 + r >= _s) & (r0 + r < _e)
                out = []
                for k in range(8):
                    x = hv[r, pl.ds(16 * k, 16)]
                    out.append(jnp.where(valid, jnp.maximum(accs2[k], x), accs2[k]))
                return tuple(out)
            return lax.fori_loop(0, RC, row, accs)

        neg = jnp.full((16,), -jnp.inf, jnp.float32)
        accs = lax.fori_loop(0, nch, chunk, (neg,) * 8)
        for k in range(8):
            stag[pl.ds(128 * gl + 16 * k, 16)] = accs[k]
    pltpu.sync_copy(stag, out_hbm.at[pl.ds(1024 * wid, 1024)])


def _sc_maxpool(h, cnt1d):
    mesh = plsc.VectorSubcoreMesh(core_axis_name="c", subcore_axis_name="s")
    out = pl.kernel(
        _maxpool_body,
        out_type=jax.ShapeDtypeStruct((G * H,), jnp.float32),
        mesh=mesh,
        compiler_params=pltpu.CompilerParams(needs_layout_passes=False),
        scratch_types=[
            pltpu.VMEM((G,), jnp.int32),
            pltpu.VMEM((272,), jnp.int32),
            pltpu.VMEM((48, H), jnp.float32),
            pltpu.VMEM((1024,), jnp.float32),
            pltpu.SemaphoreType.DMA,
        ],
    )(h, cnt1d)
    return out.reshape(G, H)


def kernel(x, edge_attr, params, edge_index, batch):
    del edge_attr
    src = edge_index[0].astype(jnp.int32)
    dst = edge_index[1].astype(jnp.int32)
    batch = batch.astype(jnp.int32)
    batch3 = batch.reshape(NB, 1, B)

    xpad = jnp.pad(x, ((0, 0), (0, H - x.shape[1])))
    lp = params['layers']
    w1pad = jnp.pad(lp[0]['W1'], ((0, H - lp[0]['W1'].shape[0]), (0, 0)))

    u, cnt = _p0(xpad, w1pad, batch3)
    cnt1d = cnt[:G, 0].astype(jnp.int32)
    zeros782 = jnp.zeros((TROWS, H), jnp.float32)

    row = lambda v: v.reshape(1, H)
    s1 = s2 = None
    for i in range(4):
        p = lp[i]
        agg = _sc_agg(u, src, dst, zeros782)
        t, s1, s2 = _bpass(u, agg, row(p['b1']), batch3)
        h, s1, s2 = _dpass(t, batch3, s1, s2, cnt, row(p['g1']), row(p['be1']),
                           row(p['a1']), p['W2'], row(p['b2']))
        if i < 3:
            n = params['norms'][i]
            u = _fpass(h, batch3, s1, s2, cnt, row(n['g']), row(n['be']),
                       row(n['a']), lp[i + 1]['W1'])

    mx = _sc_maxpool(h, cnt1d)
    out8 = _head(s1[:G], cnt[:G], mx, params['fc1_W'],
                 params['fc1_b'].reshape(1, 64),
                 jnp.pad(params['fc2_W'], ((0, 0), (0, 5))),
                 jnp.pad(params['fc2_b'], (0, 5)).reshape(1, 8))
    return out8[:, :3]


# speedup vs baseline: 1.5890x; 1.0004x over previous
import jax
import jax.numpy as jnp
from jax import lax
from jax.experimental import pallas as pl
from jax.experimental.pallas import tpu as pltpu
from jax.experimental.pallas import tpu_sc as plsc

N = 100000
E = 1600000
G = 256
H = 128
B = 1000
NB = N // B
GW = 24
GP = G + 16
EPS = 1e-5


def _onehot(bg, g0):
    return (bg[:, None] - g0 == lax.broadcasted_iota(jnp.int32, (B, GW), 1)).astype(jnp.float32)


def _wbase(bg):
    return (bg[0] // 8) * 8


def _segdot(m, t):
    return lax.dot_general(m, t, (((0,), (0,)), ((), ())),
                           precision=lax.Precision.HIGHEST,
                           preferred_element_type=jnp.float32)


def _leaky(x):
    return jnp.where(x >= 0, x, 0.01 * x)


def _p0_body(x_ref, w_ref, b3_ref, u_ref, cnt_ref):
    i = pl.program_id(0)

    @pl.when(i == 0)
    def _():
        cnt_ref[...] = jnp.zeros_like(cnt_ref)

    u_ref[...] = jnp.dot(x_ref[...], w_ref[...], preferred_element_type=jnp.float32)
    bg = b3_ref[0, 0, :]
    g0 = _wbase(bg)
    m = _onehot(bg, g0)
    cnt_ref[pl.ds(g0, GW), :] += _segdot(m, jnp.ones((B, H), jnp.float32))


def _p0(xpad, w1pad, batch3):
    return pl.pallas_call(
        _p0_body,
        grid=(NB,),
        in_specs=[
            pl.BlockSpec((B, H), lambda i: (i, 0)),
            pl.BlockSpec((H, H), lambda i: (0, 0)),
            pl.BlockSpec((1, 1, B), lambda i: (i, 0, 0)),
        ],
        out_specs=[
            pl.BlockSpec((B, H), lambda i: (i, 0)),
            pl.BlockSpec((GP, H), lambda i: (0, 0)),
        ],
        out_shape=[
            jax.ShapeDtypeStruct((N, H), jnp.float32),
            jax.ShapeDtypeStruct((GP, H), jnp.float32),
        ],
    )(xpad, w1pad, batch3)


def _bpass_body(u_ref, agg_ref, b1_ref, b3_ref, t_ref, s1_ref, s2_ref):
    i = pl.program_id(0)

    @pl.when(i == 0)
    def _():
        s1_ref[...] = jnp.zeros_like(s1_ref)
        s2_ref[...] = jnp.zeros_like(s2_ref)

    t = u_ref[...] + agg_ref[...] + b1_ref[...]
    t_ref[...] = t
    bg = b3_ref[0, 0, :]
    g0 = _wbase(bg)
    m = _onehot(bg, g0)
    s1_ref[pl.ds(g0, GW), :] += _segdot(m, t)
    s2_ref[pl.ds(g0, GW), :] += _segdot(m, t * t)


def _bpass(u, agg, b1, batch3):
    return pl.pallas_call(
        _bpass_body,
        grid=(NB,),
        in_specs=[
            pl.BlockSpec((B, H), lambda i: (i, 0)),
            pl.BlockSpec((B, H), lambda i: (i, 0)),
            pl.BlockSpec((1, H), lambda i: (0, 0)),
            pl.BlockSpec((1, 1, B), lambda i: (i, 0, 0)),
        ],
        out_specs=[
            pl.BlockSpec((B, H), lambda i: (i, 0)),
            pl.BlockSpec((GP, H), lambda i: (0, 0)),
            pl.BlockSpec((GP, H), lambda i: (0, 0)),
        ],
        out_shape=[
            jax.ShapeDtypeStruct((N, H), jnp.float32),
            jax.ShapeDtypeStruct((GP, H), jnp.float32),
            jax.ShapeDtypeStruct((GP, H), jnp.float32),
        ],
    )(u, agg, b1, batch3)


def _norm_coeffs(s1, s2, cnt, g, be, a):
    cntc = jnp.maximum(cnt, 1.0)
    mean = s1 / cntc
    var = s2 / cntc - (2.0 * a - a * a) * mean * mean
    var = jnp.maximum(var, 0.0)
    scale = g / jnp.sqrt(var + EPS)
    empty = cnt == 0.0
    scale = jnp.where(empty, 0.0, scale)
    shift = jnp.where(empty, 0.0, be - scale * a * mean)
    return scale, shift


def _dpass_body(t_ref, b3_ref, s1_ref, s2_ref, cnt_ref, g_ref, be_ref, a_ref,
                w2_ref, b2_ref, h_ref, o1_ref, o2_ref, scale_ref, shift_ref):
    i = pl.program_id(0)

    @pl.when(i == 0)
    def _():
        scale, shift = _norm_coeffs(s1_ref[...], s2_ref[...], cnt_ref[...],
                                    g_ref[...], be_ref[...], a_ref[...])
        scale_ref[...] = scale
        shift_ref[...] = shift
        o1_ref[...] = jnp.zeros_like(o1_ref)
        o2_ref[...] = jnp.zeros_like(o2_ref)

    bg = b3_ref[0, 0, :]
    g0 = _wbase(bg)
    m = _onehot(bg, g0)
    sc = jnp.dot(m, scale_ref[pl.ds(g0, GW), :], precision=lax.Precision.HIGHEST,
                 preferred_element_type=jnp.float32)
    sh = jnp.dot(m, shift_ref[pl.ds(g0, GW), :], precision=lax.Precision.HIGHEST,
                 preferred_element_type=jnp.float32)
    y = _leaky(sc * t_ref[...] + sh)
    h = jnp.dot(y, w2_ref[...], preferred_element_type=jnp.float32) + b2_ref[...]
    h_ref[...] = h
    o1_ref[pl.ds(g0, GW), :] += _segdot(m, h)
    o2_ref[pl.ds(g0, GW), :] += _segdot(m, h * h)


def _dpass(t, batch3, s1, s2, cnt, g, be, a, w2, b2):
    return pl.pallas_call(
        _dpass_body,
        grid=(NB,),
        in_specs=[
            pl.BlockSpec((B, H), lambda i: (i, 0)),
            pl.BlockSpec((1, 1, B), lambda i: (i, 0, 0)),
            pl.BlockSpec((GP, H), lambda i: (0, 0)),
            pl.BlockSpec((GP, H), lambda i: (0, 0)),
            pl.BlockSpec((GP, H), lambda i: (0, 0)),
            pl.BlockSpec((1, H), lambda i: (0, 0)),
            pl.BlockSpec((1, H), lambda i: (0, 0)),
            pl.BlockSpec((1, H), lambda i: (0, 0)),
            pl.BlockSpec((H, H), lambda i: (0, 0)),
            pl.BlockSpec((1, H), lambda i: (0, 0)),
        ],
        out_specs=[
            pl.BlockSpec((B, H), lambda i: (i, 0)),
            pl.BlockSpec((GP, H), lambda i: (0, 0)),
            pl.BlockSpec((GP, H), lambda i: (0, 0)),
        ],
        out_shape=[
            jax.ShapeDtypeStruct((N, H), jnp.float32),
            jax.ShapeDtypeStruct((GP, H), jnp.float32),
            jax.ShapeDtypeStruct((GP, H), jnp.float32),
        ],
        scratch_shapes=[
            pltpu.VMEM((GP, H), jnp.float32),
            pltpu.VMEM((GP, H), jnp.float32),
        ],
    )(t, batch3, s1, s2, cnt, g, be, a, w2, b2)


def _fpass_body(h_ref, b3_ref, s1_ref, s2_ref, cnt_ref, g_ref, be_ref, a_ref,
                w1_ref, u_ref, scale_ref, shift_ref):
    i = pl.program_id(0)

    @pl.when(i == 0)
    def _():
        scale, shift = _norm_coeffs(s1_ref[...], s2_ref[...], cnt_ref[...],
                                    g_ref[...], be_ref[...], a_ref[...])
        scale_ref[...] = scale
        shift_ref[...] = shift

    bg = b3_ref[0, 0, :]
    g0 = _wbase(bg)
    m = _onehot(bg, g0)
    sc = jnp.dot(m, scale_ref[pl.ds(g0, GW), :], precision=lax.Precision.HIGHEST,
                 preferred_element_type=jnp.float32)
    sh = jnp.dot(m, shift_ref[pl.ds(g0, GW), :], precision=lax.Precision.HIGHEST,
                 preferred_element_type=jnp.float32)
    hn = _leaky(sc * h_ref[...] + sh)
    u_ref[...] = jnp.dot(hn, w1_ref[...], preferred_element_type=jnp.float32)


def _fpass(h, batch3, s1, s2, cnt, g, be, a, w1n):
    return pl.pallas_call(
        _fpass_body,
        grid=(NB,),
        in_specs=[
            pl.BlockSpec((B, H), lambda i: (i, 0)),
            pl.BlockSpec((1, 1, B), lambda i: (i, 0, 0)),
            pl.BlockSpec((GP, H), lambda i: (0, 0)),
            pl.BlockSpec((GP, H), lambda i: (0, 0)),
            pl.BlockSpec((GP, H), lambda i: (0, 0)),
            pl.BlockSpec((1, H), lambda i: (0, 0)),
            pl.BlockSpec((1, H), lambda i: (0, 0)),
            pl.BlockSpec((1, H), lambda i: (0, 0)),
            pl.BlockSpec((H, H), lambda i: (0, 0)),
        ],
        out_specs=pl.BlockSpec((B, H), lambda i: (i, 0)),
        out_shape=jax.ShapeDtypeStruct((N, H), jnp.float32),
        scratch_shapes=[
            pltpu.VMEM((GP, H), jnp.float32),
            pltpu.VMEM((GP, H), jnp.float32),
        ],
    )(h, batch3, s1, s2, cnt, g, be, a, w1n)


def _head_body(s_ref, cnt_ref, mx_ref, w1_ref, b1_ref, w2_ref, b2_ref, o_ref):
    s = s_ref[...]
    cntc = jnp.maximum(cnt_ref[...], 1.0)
    z = jnp.concatenate([s / cntc, s, mx_ref[...]], axis=1)
    h1 = jnp.dot(z, w1_ref[...], preferred_element_type=jnp.float32) + b1_ref[...]
    h1 = _leaky(h1)
    o_ref[...] = jnp.dot(h1, w2_ref[...], preferred_element_type=jnp.float32) + b2_ref[...]


def _head(s, cnt, mx, fc1w, fc1b, fc2w8, fc2b8):
    return pl.pallas_call(
        _head_body,
        out_shape=jax.ShapeDtypeStruct((G, 8), jnp.float32),
    )(s, cnt, mx, fc1w, fc1b, fc2w8, fc2b8)


NPASS = 4
RB = 12500
RBP = 12544
TROWS = RBP // 16
ET = E // 16
CH = 2000
NVR = CH // 16
FCAP = 96


def _agg_body(u_hbm, src_hbm, dst_hbm, z_hbm, out_hbm,
              sels, seld, bsrcA, bdstA, bsrcB, bdstB, rowsA, rowsB,
              dstbuf, srcbuf, accum, gsem, ssemA, ssemB):
    core = lax.axis_index("c")
    tid = lax.axis_index("s")
    lane = lax.broadcasted_iota(jnp.int32, (16,), 0)
    pad_src = (tid * 997 + lane * 61) % N

    def fire(pos, f):
        even = lax.rem(f, 2) == 0

        @pl.when(even)
        def _():
            @pl.when(f >= 2)
            def _():
                pltpu.make_async_copy(rowsA, accum.at[bdstA], ssemA).wait()
            for k in range(FCAP // 16):
                bsrcA[pl.ds(16 * k, 16)] = sels[pl.ds(16 * k, 16)]
                bdstA[pl.ds(16 * k, 16)] = seld[pl.ds(16 * k, 16)]
            pltpu.async_copy(u_hbm.at[bsrcA], rowsA, gsem).wait()
            pltpu.async_copy(rowsA, accum.at[bdstA], ssemA, add=True)

        @pl.when(jnp.logical_not(even))
        def _():
            @pl.when(f >= 2)
            def _():
                pltpu.make_async_copy(rowsB, accum.at[bdstB], ssemB).wait()
            for k in range(FCAP // 16):
                bsrcB[pl.ds(16 * k, 16)] = sels[pl.ds(16 * k, 16)]
                bdstB[pl.ds(16 * k, 16)] = seld[pl.ds(16 * k, 16)]
            pltpu.async_copy(u_hbm.at[bsrcB], rowsB, gsem).wait()
            pltpu.async_copy(rowsB, accum.at[bdstB], ssemB, add=True)

        tl_s = sels[pl.ds(FCAP, 16)]
        tl_d = seld[pl.ds(FCAP, 16)]
        sels[pl.ds(0, 16)] = tl_s
        seld[pl.ds(0, 16)] = tl_d
        return pos - FCAP, f + 1

    for p in range(NPASS):
        bkt = 2 * p + core
        lo = bkt * RB
        spare_row = RB + tid
        pltpu.sync_copy(z_hbm, accum.at[pl.ds(TROWS * tid, TROWS)])
        plsc.subcore_barrier()

        def vreg_step(v, carry, _lo=lo, _spare=spare_row):
            pos, f = carry
            d = dstbuf[pl.ds(16 * v, 16)]
            sv = srcbuf[pl.ds(16 * v, 16)]
            m = (d >= _lo) & (d < _lo + RB)
            dl = jnp.where(m, d - _lo, _spare)
            mi = m.astype(jnp.int32)
            excl = plsc.cumsum(mi) - mi
            idx = jnp.where(m, pos + excl, 2 * FCAP)
            plsc.store_scatter(sels, [idx], sv)
            plsc.store_scatter(seld, [idx], dl)
            pos = pos + jnp.sum(mi)
            return lax.cond(pos >= FCAP, fire, lambda q, g: (q, g), pos, f)

        def chunk_step(c, carry, _vs=vreg_step):
            base = tid * ET + c * CH
            pltpu.sync_copy(dst_hbm.at[pl.ds(base, CH)], dstbuf)
            pltpu.sync_copy(src_hbm.at[pl.ds(base, CH)], srcbuf)
            return lax.fori_loop(0, NVR, _vs, carry)

        pos, f = lax.fori_loop(0, ET // CH, chunk_step, (0, 0))
        spare_v = jnp.full((16,), spare_row, jnp.int32)
        for k in range(FCAP // 16):
            sels[pl.ds(pos + 16 * k, 16)] = pad_src
            seld[pl.ds(pos + 16 * k, 16)] = spare_v
        _, f = fire(FCAP, f)
        @pl.when(lax.rem(f - 1, 2) == 0)
        def _():
            pltpu.make_async_copy(rowsA, accum.at[bdstA], ssemA).wait()
            @pl.when(f >= 2)
            def _():
                pltpu.make_async_copy(rowsB, accum.at[bdstB], ssemB).wait()
        @pl.when(lax.rem(f - 1, 2) == 1)
        def _():
            pltpu.make_async_copy(rowsB, accum.at[bdstB], ssemB).wait()
            @pl.when(f >= 2)
            def _():
                pltpu.make_async_copy(rowsA, accum.at[bdstA], ssemA).wait()
        plsc.subcore_barrier()
        r0 = TROWS * tid
        for k in range(14):
            pltpu.sync_copy(accum.at[pl.ds(r0 + 56 * k, 56)],
                            out_hbm.at[bkt, pl.ds(r0 + 56 * k, 56)])
        plsc.subcore_barrier()


def _sc_agg(u, src, dst, zeros782):
    mesh = plsc.VectorSubcoreMesh(core_axis_name="c", subcore_axis_name="s")
    out = pl.kernel(
        _agg_body,
        out_type=jax.ShapeDtypeStruct((8, RBP, H), jnp.float32),
        mesh=mesh,
        compiler_params=pltpu.CompilerParams(needs_layout_passes=False),
        scratch_types=[
            pltpu.VMEM((272,), jnp.int32),
            pltpu.VMEM((272,), jnp.int32),
            pltpu.VMEM((FCAP,), jnp.int32),
            pltpu.VMEM((FCAP,), jnp.int32),
            pltpu.VMEM((FCAP,), jnp.int32),
            pltpu.VMEM((FCAP,), jnp.int32),
            pltpu.VMEM((FCAP, H), jnp.float32),
            pltpu.VMEM((FCAP, H), jnp.float32),
            pltpu.VMEM((CH,), jnp.int32),
            pltpu.VMEM((CH,), jnp.int32),
            pltpu.VMEM_SHARED((RBP, H), jnp.float32),
            pltpu.SemaphoreType.DMA,
            pltpu.SemaphoreType.DMA,
            pltpu.SemaphoreType.DMA,
        ],
    )(u, src, dst, zeros782)
    return out[:, :RB, :].reshape(N, H)


def _maxpool_body(h_hbm, cnt_hbm, out_hbm, cntv, offs, hv, stag, sem):
    core = lax.axis_index("c")
    tid = lax.axis_index("s")
    wid = core * 16 + tid
    lane = lax.broadcasted_iota(jnp.int32, (16,), 0)
    pltpu.sync_copy(cnt_hbm, cntv)
    def off_step(k, carry):
        v = cntv[pl.ds(16 * k, 16)]
        c = plsc.cumsum(v)
        offs[pl.ds(16 * k, 16)] = c - v + carry
        return carry + jnp.sum(v)

    total = lax.fori_loop(0, 16, off_step, jnp.int32(0))
    offs[pl.ds(256, 16)] = jnp.full((16,), total, jnp.int32)
    ov = offs[pl.ds(8 * wid, 16)]

    def pick(j):
        return jnp.sum(jnp.where(lane == j, ov, 0))

    RC = 48
    for gl in range(8):
        start = pick(gl)
        end = pick(gl + 1)
        s8 = (start // 8) * 8
        nch = (end - s8 + RC - 9) // (RC - 8) + 1

        def chunk(j, accs, _s=start, _e=end, _s8=s8):
            r0 = jnp.minimum(_s8 + (RC - 8) * j, N - RC)
            pltpu.async_copy(h_hbm.at[pl.ds(r0, RC)], hv, sem).wait()
            def row(r, accs2):
                valid = (r0 + r >= _s) & (r0 + r < _e)
                out = []
                for k in range(8):
                    x = hv[r, pl.ds(16 * k, 16)]
                    out.append(jnp.where(valid, jnp.maximum(accs2[k], x), accs2[k]))
                return tuple(out)
            return lax.fori_loop(0, RC, row, accs)

        neg = jnp.full((16,), -jnp.inf, jnp.float32)
        accs = lax.fori_loop(0, nch, chunk, (neg,) * 8)
        for k in range(8):
            stag[pl.ds(128 * gl + 16 * k, 16)] = accs[k]
    pltpu.sync_copy(stag, out_hbm.at[pl.ds(1024 * wid, 1024)])


def _sc_maxpool(h, cnt1d):
    mesh = plsc.VectorSubcoreMesh(core_axis_name="c", subcore_axis_name="s")
    out = pl.kernel(
        _maxpool_body,
        out_type=jax.ShapeDtypeStruct((G * H,), jnp.float32),
        mesh=mesh,
        compiler_params=pltpu.CompilerParams(needs_layout_passes=False),
        scratch_types=[
            pltpu.VMEM((G,), jnp.int32),
            pltpu.VMEM((272,), jnp.int32),
            pltpu.VMEM((48, H), jnp.float32),
            pltpu.VMEM((1024,), jnp.float32),
            pltpu.SemaphoreType.DMA,
        ],
    )(h, cnt1d)
    return out.reshape(G, H)


def kernel(x, edge_attr, params, edge_index, batch):
    del edge_attr
    src = edge_index[0].astype(jnp.int32)
    dst = edge_index[1].astype(jnp.int32)
    batch = batch.astype(jnp.int32)
    batch3 = batch.reshape(NB, 1, B)

    xpad = jnp.pad(x, ((0, 0), (0, H - x.shape[1])))
    lp = params['layers']
    w1pad = jnp.pad(lp[0]['W1'], ((0, H - lp[0]['W1'].shape[0]), (0, 0)))

    u, cnt = _p0(xpad, w1pad, batch3)
    cnt1d = cnt[:G, 0].astype(jnp.int32)
    zeros782 = jnp.zeros((TROWS, H), jnp.float32)

    row = lambda v: v.reshape(1, H)
    s1 = s2 = None
    for i in range(4):
        p = lp[i]
        agg = _sc_agg(u, src, dst, zeros782)
        t, s1, s2 = _bpass(u, agg, row(p['b1']), batch3)
        h, s1, s2 = _dpass(t, batch3, s1, s2, cnt, row(p['g1']), row(p['be1']),
                           row(p['a1']), p['W2'], row(p['b2']))
        if i < 3:
            n = params['norms'][i]
            u = _fpass(h, batch3, s1, s2, cnt, row(n['g']), row(n['be']),
                       row(n['a']), lp[i + 1]['W1'])

    mx = _sc_maxpool(h, cnt1d)
    out8 = _head(s1[:G], cnt[:G], mx, params['fc1_W'],
                 params['fc1_b'].reshape(1, 64),
                 jnp.pad(params['fc2_W'], ((0, 0), (0, 5))),
                 jnp.pad(params['fc2_b'], (0, 5)).reshape(1, 8))
    return out8[:, :3]
